# baseline (speedup 1.0000x reference)
import jax
import jax.numpy as jnp
from jax import lax
from jax.experimental import pallas as pl
from jax.experimental.pallas import tpu as pltpu
from jax.experimental.pallas import tpu_sc as plsc

B = 4096
L = 50
D = 64
V = 100000
LANES = 16
DW = D // 2

NC, NS = 2, 16
NW = NC * NS
BPW = B // NW
CH = 2
IDX_RAW = CH * L
IDX_PAD = 104
NSTREAM = BPW // CH
NBUF = 4
IDXBUF = NSTREAM * IDX_PAD + 16

def _split_bf16_pair(w):
    hi_mask = jnp.full((LANES,), -65536, dtype=jnp.int32)
    even = lax.bitcast_convert_type(lax.shift_left(w, 16), jnp.float32)
    odd = lax.bitcast_convert_type(lax.bitwise_and(w, hi_mask), jnp.float32)
    return even, odd


_GATHER_DNUMS = lax.GatherDimensionNumbers(
    offset_dims=(), collapsed_slice_dims=(0,), start_index_map=(0,)
)


def _lane_shuffle(v, idx):
    return lax.gather(
        v,
        idx[:, None],
        dimension_numbers=_GATHER_DNUMS,
        slice_sizes=(1,),
        mode=lax.GatherScatterMode.PROMISE_IN_BOUNDS,
    )


def _allsum16(v):
    lane = lax.iota(jnp.int32, LANES)
    for s in (1, 2, 4, 8):
        v = v + _lane_shuffle(v, jnp.bitwise_xor(lane, s))
    return v


def _rsqrt16(sv):
    yi = jnp.full((LANES,), 0x5F3759DF, dtype=jnp.int32) - (
        lax.shift_right_logical(lax.bitcast_convert_type(sv, jnp.int32), 1)
    )
    y = lax.bitcast_convert_type(yi, jnp.float32)
    half = sv * jnp.float32(0.5)
    for _ in range(3):
        y = y * (jnp.float32(1.5) - half * y * y)
    return y


def _sc_body(idx_hbm, table_hbm, out_hbm, stage_v, idx_v,
             buf0, buf1, buf2, buf3, acc_v, out_v, sem0, sem1, sem2, sem3):
    wid = lax.axis_index("s") * NC + lax.axis_index("c")
    bufs = (buf0, buf1, buf2, buf3)
    sems = (sem0, sem1, sem2, sem3)

    pltpu.sync_copy(idx_hbm.at[wid], stage_v)

    z16 = jnp.zeros((LANES,), jnp.int32)

    def repack_body(j, carry):
        dst = j * IDX_PAD
        src = j * IDX_RAW
        idx_v[pl.ds(dst + 96, LANES)] = z16
        for k in range(0, IDX_RAW - LANES + 1, LANES):
            idx_v[pl.ds(dst + k, LANES)] = stage_v[pl.ds(src + k, LANES)]
        t = IDX_RAW - LANES
        idx_v[pl.ds(dst + t, LANES)] = stage_v[pl.ds(src + t, LANES)]
        return carry

    lax.fori_loop(0, NSTREAM, repack_body, 0, unroll=False)

    def start(j, b):
        pltpu.async_copy(
            table_hbm.at[idx_v.at[pl.ds(j * IDX_PAD, IDX_PAD)]],
            bufs[b], sems[b])

    def wait(j, b):
        pltpu.make_async_copy(
            table_hbm.at[idx_v.at[pl.ds(j * IDX_PAD, IDX_PAD)]],
            bufs[b], sems[b]).wait()

    def accumulate(j, buf):
        for c in range(CH):
            base = c * L

            def acc_l(l, carry):
                e0, o0, e1, o1 = carry
                w0 = buf[base + l, pl.ds(0, LANES)]
                w1 = buf[base + l, pl.ds(LANES, LANES)]
                ev0, od0 = _split_bf16_pair(w0)
                ev1, od1 = _split_bf16_pair(w1)
                return e0 + ev0, o0 + od0, e1 + ev1, o1 + od1

            zf = jnp.zeros((LANES,), jnp.float32)
            e0, o0, e1, o1 = lax.fori_loop(
                0, L, acc_l, (zf, zf, zf, zf), unroll=10)
            r = j * CH + c
            acc_v[r, pl.ds(0, LANES)] = e0
            acc_v[r, pl.ds(LANES, LANES)] = o0
            acc_v[r, pl.ds(2 * LANES, LANES)] = e1
            acc_v[r, pl.ds(3 * LANES, LANES)] = o1

    for b in range(NBUF):
        start(b, b)

    def stream_body(jj, carry):
        for b in range(NBUF):
            j = jj * NBUF + b
            wait(j, b)
            accumulate(j, bufs[b])
            nxt = jnp.minimum(j + NBUF, NSTREAM - 1)

            @pl.when(j + NBUF < NSTREAM)
            def _():
                start(nxt, b)
        return carry

    lax.fori_loop(0, NSTREAM // NBUF, stream_body, 0, unroll=False)

    lane = lax.iota(jnp.int32, LANES)
    half_lo = lax.shift_right_logical(lane, 1)
    half_hi = half_lo + jnp.int32(8)
    is_even = lax.eq(jnp.bitwise_and(lane, 1), jnp.int32(0))

    def interleave(e, o, hi):
        idx = half_hi if hi else half_lo
        return jnp.where(is_even, _lane_shuffle(e, idx), _lane_shuffle(o, idx))

    def norm_body(r, carry):
        e0 = acc_v[r, pl.ds(0, LANES)]
        o0 = acc_v[r, pl.ds(LANES, LANES)]
        e1 = acc_v[r, pl.ds(2 * LANES, LANES)]
        o1 = acc_v[r, pl.ds(3 * LANES, LANES)]
        v = e0 * e0 + o0 * o0 + e1 * e1 + o1 * o1
        sv = jnp.maximum(_allsum16(v), jnp.float32(1e-24))
        y = _rsqrt16(sv)
        e0, o0, e1, o1 = e0 * y, o0 * y, e1 * y, o1 * y
        out_v[r, pl.ds(0, LANES)] = interleave(e0, o0, False)
        out_v[r, pl.ds(LANES, LANES)] = interleave(e0, o0, True)
        out_v[r, pl.ds(2 * LANES, LANES)] = interleave(e1, o1, False)
        out_v[r, pl.ds(3 * LANES, LANES)] = interleave(e1, o1, True)
        return carry

    lax.fori_loop(0, BPW, norm_body, 0, unroll=False)

    pltpu.sync_copy(out_v, out_hbm.at[pl.ds(wid * BPW, BPW)])


@jax.jit
def _mean_embed(idx_blocks, table_words):
    mesh = plsc.VectorSubcoreMesh(core_axis_name="c", subcore_axis_name="s")
    f = pl.kernel(
        _sc_body,
        out_type=jax.ShapeDtypeStruct((B, D), jnp.float32),
        mesh=mesh,
        compiler_params=pltpu.CompilerParams(use_tc_tiling_on_sc=False),
        scratch_types=(
            [pltpu.VMEM((BPW * L,), jnp.int32),
             pltpu.VMEM((IDXBUF,), jnp.int32)]
            + [pltpu.VMEM((IDX_PAD, DW), jnp.int32) for _ in range(NBUF)]
            + [pltpu.VMEM((BPW, D), jnp.float32),
               pltpu.VMEM((BPW, D), jnp.float32)]
            + [pltpu.SemaphoreType.DMA for _ in range(NBUF)]
        ),
    )
    return f(idx_blocks, table_words)


def kernel(text_idxs, text_len, embedding_table):
    del text_len
    idx = text_idxs.astype(jnp.int32).reshape(NW, BPW * L)
    table_words = lax.bitcast_convert_type(
        embedding_table.astype(jnp.bfloat16).reshape(V, DW, 2), jnp.int32)
    return _mean_embed(idx, table_words)
